# Initial kernel scaffold; baseline (speedup 1.0000x reference)
import jax
import jax.numpy as jnp
from jax.experimental import pallas as pl


def kernel(x, table):
    raise NotImplementedError("write your pallas kernel here")


# speedup vs baseline: 1.8367x; 1.8367x over previous
import functools

import jax
import jax.numpy as jnp
from jax import lax
from jax.experimental import pallas as pl
from jax.experimental.pallas import tpu as pltpu
from jax.experimental.pallas import tpu_sc as plsc

EMBED = 64
CHUNK = 128
NBUF = 2


@functools.cache
def _build_gather(n_rows: int):
    info = plsc.get_sparse_core_info()
    nw = info.num_cores * info.num_subcores
    rows_per_w = n_rows // nw
    n_chunks = rows_per_w // CHUNK
    assert rows_per_w % CHUNK == 0 and n_rows % nw == 0

    mesh = plsc.VectorSubcoreMesh(core_axis_name="c", subcore_axis_name="s")

    @functools.partial(
        pl.kernel,
        mesh=mesh,
        out_type=jax.ShapeDtypeStruct((n_rows, EMBED), jnp.float32),
        scratch_types=[
            pltpu.VMEM((n_chunks, CHUNK), jnp.int32),
            pltpu.VMEM((NBUF, CHUNK, EMBED), jnp.float32),
            pltpu.SemaphoreType.DMA,
            pltpu.SemaphoreType.DMA,
        ],
        compiler_params=pltpu.CompilerParams(use_tc_tiling_on_sc=False),
    )
    def gather_kernel(idx_hbm, table_hbm, out_hbm, idx_v, rows_v, sem0, sem1):
        sems = (sem0, sem1)
        wid = lax.axis_index("s") * info.num_cores + lax.axis_index("c")
        chunk0 = wid * n_chunks
        base = wid * rows_per_w

        pltpu.sync_copy(idx_hbm.at[pl.ds(chunk0, n_chunks)], idx_v)

        for b in range(NBUF):
            pltpu.async_copy(table_hbm.at[idx_v.at[b]], rows_v.at[b], sems[b])

        @pl.loop(0, n_chunks, step=NBUF)
        def _(g):
            for b in range(NBUF):
                j = g + b
                pltpu.make_async_copy(
                    table_hbm.at[idx_v.at[j]], rows_v.at[b], sems[b]
                ).wait()
                pltpu.sync_copy(
                    rows_v.at[b], out_hbm.at[pl.ds(base + j * CHUNK, CHUNK)]
                )
                nxt = j + NBUF

                @pl.when(nxt < n_chunks)
                def _():
                    pltpu.async_copy(
                        table_hbm.at[idx_v.at[nxt]], rows_v.at[b], sems[b]
                    )

    return gather_kernel


@jax.jit
def kernel(x, table):
    batch, hist = x.shape
    n_rows = batch * hist
    idx2d = x.reshape(n_rows // CHUNK, CHUNK).astype(jnp.int32)
    out = _build_gather(n_rows)(idx2d, table)
    return out.reshape(batch, hist, EMBED)


# speedup vs baseline: 1.8750x; 1.0209x over previous
import functools

import jax
import jax.numpy as jnp
from jax import lax
from jax.experimental import pallas as pl
from jax.experimental.pallas import tpu as pltpu
from jax.experimental.pallas import tpu_sc as plsc

EMBED = 64
CHUNK = 256
NBUF = 2


@functools.cache
def _build_gather(n_rows: int):
    info = plsc.get_sparse_core_info()
    nw = info.num_cores * info.num_subcores
    rows_per_w = n_rows // nw
    n_chunks = rows_per_w // CHUNK
    assert rows_per_w % CHUNK == 0 and n_rows % nw == 0

    mesh = plsc.VectorSubcoreMesh(core_axis_name="c", subcore_axis_name="s")

    @functools.partial(
        pl.kernel,
        mesh=mesh,
        out_type=jax.ShapeDtypeStruct((n_rows, EMBED), jnp.float32),
        scratch_types=[
            pltpu.VMEM((n_chunks, CHUNK), jnp.int32),
            pltpu.VMEM((NBUF, CHUNK, EMBED), jnp.float32),
            pltpu.SemaphoreType.DMA,
            pltpu.SemaphoreType.DMA,
        ],
        compiler_params=pltpu.CompilerParams(use_tc_tiling_on_sc=False),
    )
    def gather_kernel(idx_hbm, table_hbm, out_hbm, idx_v, rows_v, sem0, sem1):
        sems = (sem0, sem1)
        wid = lax.axis_index("s") * info.num_cores + lax.axis_index("c")
        chunk0 = wid * n_chunks
        base = wid * rows_per_w

        pltpu.sync_copy(idx_hbm.at[pl.ds(chunk0, n_chunks)], idx_v)

        for b in range(NBUF):
            pltpu.async_copy(table_hbm.at[idx_v.at[b]], rows_v.at[b], sems[b])

        @pl.loop(0, n_chunks, step=NBUF)
        def _(g):
            for b in range(NBUF):
                j = g + b
                pltpu.make_async_copy(
                    table_hbm.at[idx_v.at[j]], rows_v.at[b], sems[b]
                ).wait()
                pltpu.sync_copy(
                    rows_v.at[b], out_hbm.at[pl.ds(base + j * CHUNK, CHUNK)]
                )
                nxt = j + NBUF

                @pl.when(nxt < n_chunks)
                def _():
                    pltpu.async_copy(
                        table_hbm.at[idx_v.at[nxt]], rows_v.at[b], sems[b]
                    )

    return gather_kernel


@jax.jit
def kernel(x, table):
    batch, hist = x.shape
    n_rows = batch * hist
    idx2d = x.reshape(n_rows // CHUNK, CHUNK).astype(jnp.int32)
    out = _build_gather(n_rows)(idx2d, table)
    return out.reshape(batch, hist, EMBED)


# speedup vs baseline: 1.8758x; 1.0004x over previous
import functools

import jax
import jax.numpy as jnp
from jax import lax
from jax.experimental import pallas as pl
from jax.experimental.pallas import tpu as pltpu
from jax.experimental.pallas import tpu_sc as plsc

EMBED = 64
CHUNK = 256
NBUF = 4


@functools.cache
def _build_gather(n_rows: int):
    info = plsc.get_sparse_core_info()
    nw = info.num_cores * info.num_subcores
    rows_per_w = n_rows // nw
    n_chunks = rows_per_w // CHUNK
    assert rows_per_w % CHUNK == 0 and n_rows % nw == 0

    mesh = plsc.VectorSubcoreMesh(core_axis_name="c", subcore_axis_name="s")

    @functools.partial(
        pl.kernel,
        mesh=mesh,
        out_type=jax.ShapeDtypeStruct((n_rows, EMBED), jnp.float32),
        scratch_types=[
            pltpu.VMEM((n_chunks, CHUNK), jnp.int32),
            pltpu.VMEM((NBUF, CHUNK, EMBED), jnp.float32),
        ] + [pltpu.SemaphoreType.DMA] * NBUF,
        compiler_params=pltpu.CompilerParams(use_tc_tiling_on_sc=False),
    )
    def gather_kernel(idx_hbm, table_hbm, out_hbm, idx_v, rows_v, *sems):
        wid = lax.axis_index("s") * info.num_cores + lax.axis_index("c")
        chunk0 = wid * n_chunks
        base = wid * rows_per_w

        pltpu.sync_copy(idx_hbm.at[pl.ds(chunk0, n_chunks)], idx_v)

        for b in range(NBUF):
            pltpu.async_copy(table_hbm.at[idx_v.at[b]], rows_v.at[b], sems[b])

        @pl.loop(0, n_chunks, step=NBUF)
        def _(g):
            for b in range(NBUF):
                j = g + b
                pltpu.make_async_copy(
                    table_hbm.at[idx_v.at[j]], rows_v.at[b], sems[b]
                ).wait()
                pltpu.sync_copy(
                    rows_v.at[b], out_hbm.at[pl.ds(base + j * CHUNK, CHUNK)]
                )
                nxt = j + NBUF

                @pl.when(nxt < n_chunks)
                def _():
                    pltpu.async_copy(
                        table_hbm.at[idx_v.at[nxt]], rows_v.at[b], sems[b]
                    )

    return gather_kernel


@jax.jit
def kernel(x, table):
    batch, hist = x.shape
    n_rows = batch * hist
    idx2d = x.reshape(n_rows // CHUNK, CHUNK).astype(jnp.int32)
    out = _build_gather(n_rows)(idx2d, table)
    return out.reshape(batch, hist, EMBED)
